# baseline (speedup 1.0000x reference)
import functools

import jax
import jax.numpy as jnp
from jax import lax
from jax.experimental import pallas as pl
from jax.experimental.pallas import tpu as pltpu
from jax.experimental.pallas import tpu_sc as plsc

VOCAB = 1000000
D_MODEL = 64

_info = plsc.get_sparse_core_info()
_NC, _NS = _info.num_cores, _info.num_subcores
_NW = _NC * _NS

_C = 256
_TROWS = 24
_TPITCH = 136
_TBYTES = 8 * 16 * 128 * 4


def _make_gather(B: int, n_i: int):
    assert B % (_NW * _C * 2) == 0 and n_i % _C == 0
    b_per_w = B // _NW
    n_chunks = b_per_w // _C
    n_pairs = n_chunks // 2
    it_per_i = n_i // 128
    mesh = plsc.VectorSubcoreMesh(core_axis_name="c", subcore_axis_name="s")

    @functools.partial(
        pl.kernel,
        out_type=jax.ShapeDtypeStruct((B // 128 * 64, 128), jnp.float32),
        mesh=mesh,
        scratch_types=[
            pltpu.VMEM((b_per_w,), jnp.int32),
            pltpu.VMEM((_C, D_MODEL), jnp.float32),
            pltpu.VMEM((_C, D_MODEL), jnp.float32),
            pltpu.VMEM((8, _TROWS, _TPITCH), jnp.float32),
            pltpu.VMEM((8, _TROWS, _TPITCH), jnp.float32),
            pltpu.SemaphoreType.DMA,
            pltpu.SemaphoreType.DMA,
            pltpu.SemaphoreType.DMA,
            pltpu.SemaphoreType.DMA,
        ],
        compiler_params=pltpu.CompilerParams(
            use_tc_tiling_on_sc=False, needs_layout_passes=False
        ),
    )
    def gather_kernel(idx_hbm, table_hbm, out_hbm, idx_v, rb_a, rb_b, t_a, t_b,
                      gsem_a, gsem_b, ssem_a, ssem_b):
        wid = lax.axis_index("s") * _NC + lax.axis_index("c")
        base = wid * b_per_w

        def start_gather(c, rb, gsem):
            pltpu.async_copy(
                table_hbm.at[idx_v.at[pl.ds(c * _C, _C)]], rb, gsem
            )

        def wait_gather(rb, gsem):
            pltpu.make_async_copy(
                table_hbm.at[idx_v.at[pl.ds(0, _C)]], rb, gsem
            ).wait()

        lanes = lax.iota(jnp.int32, 16)
        kg_vs = [lanes // 8 + 2 * q for q in range(4)]
        ks_v = lanes % 8

        def transpose(rb, t):
            def row(c, carry):
                it = c // 128
                il = jnp.full((16,), c - it * 128, jnp.int32)
                r_v = ks_v + it * 8
                for q in range(4):
                    v = rb[c, pl.ds(q * 16, 16)]
                    plsc.store_scatter(t, [kg_vs[q], r_v, il], v)
                return carry

            lax.fori_loop(0, _C, row, 0)

        def start_store(c, t, ssem):
            n0 = base + c * _C
            j = n0 // n_i
            it0 = (n0 - j * n_i) // 128
            for kg in range(8):
                r0 = (j * 8 + kg) * it_per_i * 8 + it0 * 8
                pltpu.async_copy(
                    t.at[kg, pl.ds(0, 16), pl.ds(0, 128)],
                    out_hbm.at[pl.ds(r0, 16)],
                    ssem,
                )

        def wait_store(t, ssem):
            pltpu.make_async_copy(
                t.at[0, pl.ds(0, 16), pl.ds(0, 128)],
                out_hbm.at[pl.ds(0, 16)],
                ssem,
            ).wait()

        def wait_stores(t, ssem):
            for kg in range(8):
                wait_store(t, ssem)

        def do_pair(c, issue_next, wait_prev):
            wait_gather(rb_a, gsem_a)
            if wait_prev:
                wait_stores(t_a, ssem_a)
            transpose(rb_a, t_a)
            if issue_next:
                start_gather(c + 2, rb_a, gsem_a)
            start_store(c, t_a, ssem_a)
            wait_gather(rb_b, gsem_b)
            if wait_prev:
                wait_stores(t_b, ssem_b)
            transpose(rb_b, t_b)
            if issue_next:
                start_gather(c + 3, rb_b, gsem_b)
            start_store(c + 1, t_b, ssem_b)

        pltpu.sync_copy(idx_hbm.at[pl.ds(base, b_per_w)], idx_v)

        start_gather(0, rb_a, gsem_a)
        start_gather(1, rb_b, gsem_b)
        do_pair(0, True, False)

        def outer(p, carry):
            do_pair(p * 2, True, True)
            return carry

        lax.fori_loop(1, n_pairs - 1, outer, 0)
        do_pair((n_pairs - 1) * 2, False, True)
        wait_stores(t_a, ssem_a)
        wait_stores(t_b, ssem_b)

    return gather_kernel


_VB = 8192


def _tc_transpose(v: int, d: int):
    def body(in_ref, out_ref):
        eye = (
            lax.broadcasted_iota(jnp.int32, (d, d), 0)
            == lax.broadcasted_iota(jnp.int32, (d, d), 1)
        ).astype(jnp.float32)
        out_ref[...] = lax.dot_general(
            in_ref[...],
            eye,
            dimension_numbers=(((0,), (0,)), ((), ())),
            preferred_element_type=jnp.float32,
        )

    grid = (v + _VB - 1) // _VB
    return pl.pallas_call(
        body,
        grid=(grid,),
        in_specs=[pl.BlockSpec((d, _VB), lambda i: (0, i))],
        out_specs=pl.BlockSpec((_VB, d), lambda i: (i, 0)),
        out_shape=jax.ShapeDtypeStruct((v, d), jnp.float32),
    )


@jax.jit
def kernel(x, table):
    n_seq, n_pos = x.shape
    n_i, n_j = n_seq, n_pos
    flat = x.T.reshape(-1).astype(jnp.int32)
    table_rm = _tc_transpose(*table.shape)(table.T)
    out = _make_gather(flat.shape[0], n_i)(flat, table_rm)
    out = out.reshape(n_j, 8, n_i // 128, 8, 128)
    return out.transpose(2, 4, 0, 1, 3).reshape(n_i, n_j, D_MODEL)


# speedup vs baseline: 1.0561x; 1.0561x over previous
import functools

import jax
import jax.numpy as jnp
from jax import lax
from jax.experimental import pallas as pl
from jax.experimental.pallas import tpu as pltpu
from jax.experimental.pallas import tpu_sc as plsc

VOCAB = 1000000
D_MODEL = 64

_info = plsc.get_sparse_core_info()
_NC, _NS = _info.num_cores, _info.num_subcores
_NW = _NC * _NS

_C = 256
_TROWS = 24
_TPITCH = 136
_TBYTES = 8 * 16 * 128 * 4


def _make_gather(B: int, n_i: int):
    assert B % (_NW * _C * 2) == 0 and n_i % _C == 0
    b_per_w = B // _NW
    n_chunks = b_per_w // _C
    n_pairs = n_chunks // 2
    it_per_i = n_i // 128
    mesh = plsc.VectorSubcoreMesh(core_axis_name="c", subcore_axis_name="s")

    @functools.partial(
        pl.kernel,
        out_type=jax.ShapeDtypeStruct((B // 128 * 64, 128), jnp.float32),
        mesh=mesh,
        scratch_types=[
            pltpu.VMEM((b_per_w,), jnp.int32),
            pltpu.VMEM((_C, D_MODEL), jnp.float32),
            pltpu.VMEM((_C, D_MODEL), jnp.float32),
            pltpu.VMEM((8, _TROWS, _TPITCH), jnp.float32),
            pltpu.VMEM((8, _TROWS, _TPITCH), jnp.float32),
            pltpu.SemaphoreType.DMA,
            pltpu.SemaphoreType.DMA,
            pltpu.SemaphoreType.DMA,
            pltpu.SemaphoreType.DMA,
        ],
        compiler_params=pltpu.CompilerParams(
            use_tc_tiling_on_sc=False, needs_layout_passes=False
        ),
    )
    def gather_kernel(idx_hbm, table_hbm, out_hbm, idx_v, rb_a, rb_b, t_a, t_b,
                      gsem_a, gsem_b, ssem_a, ssem_b):
        wid = lax.axis_index("s") * _NC + lax.axis_index("c")
        base = wid * b_per_w

        def start_gather(c, rb, gsem):
            pltpu.async_copy(
                table_hbm.at[idx_v.at[pl.ds(c * _C, _C)]], rb, gsem
            )

        def wait_gather(rb, gsem):
            pltpu.make_async_copy(
                table_hbm.at[idx_v.at[pl.ds(0, _C)]], rb, gsem
            ).wait()

        lanes = lax.iota(jnp.int32, 16)
        kg_vs = [lanes // 8 + 2 * q for q in range(4)]
        ks_v = lanes % 8

        def transpose(rb, t):
            def row(c, carry):
                it = c // 128
                il = jnp.full((16,), c - it * 128, jnp.int32)
                r_v = ks_v + it * 8
                for q in range(4):
                    v = rb[c, pl.ds(q * 16, 16)]
                    plsc.store_scatter(t, [kg_vs[q], r_v, il], v)
                return carry

            lax.fori_loop(0, _C, row, 0)

        def start_store(c, t, ssem):
            n0 = base + c * _C
            j = n0 // n_i
            it0 = (n0 - j * n_i) // 128
            for kg in range(8):
                r0 = (j * 8 + kg) * it_per_i * 8 + it0 * 8
                pltpu.async_copy(
                    t.at[kg, pl.ds(0, 16), pl.ds(0, 128)],
                    out_hbm.at[pl.ds(r0, 16)],
                    ssem,
                )

        def wait_store(t, ssem):
            pltpu.make_async_copy(
                t.at[0, pl.ds(0, 16), pl.ds(0, 128)],
                out_hbm.at[pl.ds(0, 16)],
                ssem,
            ).wait()

        def wait_stores(t, ssem):
            for kg in range(8):
                wait_store(t, ssem)

        def do_pair(c, issue_next, wait_prev):
            wait_gather(rb_a, gsem_a)
            if wait_prev:
                wait_stores(t_a, ssem_a)
            transpose(rb_a, t_a)
            if issue_next:
                start_gather(c + 2, rb_a, gsem_a)
            start_store(c, t_a, ssem_a)
            wait_gather(rb_b, gsem_b)
            if wait_prev:
                wait_stores(t_b, ssem_b)
            transpose(rb_b, t_b)
            if issue_next:
                start_gather(c + 3, rb_b, gsem_b)
            start_store(c + 1, t_b, ssem_b)

        pltpu.sync_copy(idx_hbm.at[pl.ds(base, b_per_w)], idx_v)

        start_gather(0, rb_a, gsem_a)
        start_gather(1, rb_b, gsem_b)
        do_pair(0, True, False)

        def outer(p, carry):
            do_pair(p * 2, True, True)
            return carry

        lax.fori_loop(1, n_pairs - 1, outer, 0)
        do_pair((n_pairs - 1) * 2, False, True)
        wait_stores(t_a, ssem_a)
        wait_stores(t_b, ssem_b)

    return gather_kernel


@jax.jit
def kernel(x, table):
    n_seq, n_pos = x.shape
    n_i, n_j = n_seq, n_pos
    flat = x.T.reshape(-1).astype(jnp.int32)
    out = _make_gather(flat.shape[0], n_i)(flat, table)
    out = out.reshape(n_j, 8, n_i // 128, 8, 128)
    return out.transpose(2, 4, 0, 1, 3).reshape(n_i, n_j, D_MODEL)


# speedup vs baseline: 1.0716x; 1.0147x over previous
import functools

import jax
import jax.numpy as jnp
from jax import lax
from jax.experimental import pallas as pl
from jax.experimental.pallas import tpu as pltpu
from jax.experimental.pallas import tpu_sc as plsc

VOCAB = 1000000
D_MODEL = 64

_info = plsc.get_sparse_core_info()
_NC, _NS = _info.num_cores, _info.num_subcores
_NW = _NC * _NS

_C = 256
_TROWS = 24
_TPITCH = 136
_TBYTES = 8 * 16 * 128 * 4


def _make_gather(B: int, n_i: int):
    assert B % (_NW * _C * 2) == 0 and n_i % _C == 0
    b_per_w = B // _NW
    n_chunks = b_per_w // _C
    n_pairs = n_chunks // 2
    it_per_i = n_i // 128
    mesh = plsc.VectorSubcoreMesh(core_axis_name="c", subcore_axis_name="s")

    @functools.partial(
        pl.kernel,
        out_type=jax.ShapeDtypeStruct((B // 128 * 64, 128), jnp.float32),
        mesh=mesh,
        scratch_types=[
            pltpu.VMEM((b_per_w,), jnp.int32),
            pltpu.VMEM((_C, D_MODEL), jnp.float32),
            pltpu.VMEM((_C, D_MODEL), jnp.float32),
            pltpu.VMEM((8, _TROWS, _TPITCH), jnp.float32),
            pltpu.VMEM((8, _TROWS, _TPITCH), jnp.float32),
            pltpu.SemaphoreType.DMA,
            pltpu.SemaphoreType.DMA,
            pltpu.SemaphoreType.DMA,
            pltpu.SemaphoreType.DMA,
        ],
        compiler_params=pltpu.CompilerParams(
            use_tc_tiling_on_sc=False, needs_layout_passes=False
        ),
    )
    def gather_kernel(idx_hbm, table_hbm, out_hbm, idx_v, rb_a, rb_b, t_a, t_b,
                      gsem_a, gsem_b, ssem_a, ssem_b):
        wid = lax.axis_index("s") * _NC + lax.axis_index("c")
        base = wid * b_per_w

        def start_gather(c, rb, gsem):
            pltpu.async_copy(
                table_hbm.at[idx_v.at[pl.ds(c * _C, _C)]], rb, gsem
            )

        def wait_gather(rb, gsem):
            pltpu.make_async_copy(
                table_hbm.at[idx_v.at[pl.ds(0, _C)]], rb, gsem
            ).wait()

        lanes = lax.iota(jnp.int32, 16)
        kg_vs = [lanes // 8 + 2 * q for q in range(4)]
        ks_v = lanes % 8

        def transpose(rb, t):
            for it in range(_C // 128):
                r_v = ks_v + it * 8

                def rows(cl, carry):
                    for dc in range(2):
                        c = cl * 2 + dc
                        il = jnp.full((16,), c, jnp.int32)
                        for q in range(4):
                            v = rb[it * 128 + c, pl.ds(q * 16, 16)]
                            plsc.store_scatter(t, [kg_vs[q], r_v, il], v)
                    return carry

                lax.fori_loop(0, 64, rows, 0)

        def start_store(c, t, ssem):
            n0 = base + c * _C
            j = n0 // n_i
            it0 = (n0 - j * n_i) // 128
            for kg in range(8):
                r0 = (j * 8 + kg) * it_per_i * 8 + it0 * 8
                pltpu.async_copy(
                    t.at[kg, pl.ds(0, 16), pl.ds(0, 128)],
                    out_hbm.at[pl.ds(r0, 16)],
                    ssem,
                )

        def wait_store(t, ssem):
            pltpu.make_async_copy(
                t.at[0, pl.ds(0, 16), pl.ds(0, 128)],
                out_hbm.at[pl.ds(0, 16)],
                ssem,
            ).wait()

        def wait_stores(t, ssem):
            for kg in range(8):
                wait_store(t, ssem)

        def do_pair(c, issue_next, wait_prev):
            wait_gather(rb_a, gsem_a)
            if wait_prev:
                wait_stores(t_a, ssem_a)
            transpose(rb_a, t_a)
            if issue_next:
                start_gather(c + 2, rb_a, gsem_a)
            start_store(c, t_a, ssem_a)
            wait_gather(rb_b, gsem_b)
            if wait_prev:
                wait_stores(t_b, ssem_b)
            transpose(rb_b, t_b)
            if issue_next:
                start_gather(c + 3, rb_b, gsem_b)
            start_store(c + 1, t_b, ssem_b)

        pltpu.sync_copy(idx_hbm.at[pl.ds(base, b_per_w)], idx_v)

        start_gather(0, rb_a, gsem_a)
        start_gather(1, rb_b, gsem_b)
        do_pair(0, True, False)

        def outer(p, carry):
            do_pair(p * 2, True, True)
            return carry

        lax.fori_loop(1, n_pairs - 1, outer, 0)
        do_pair((n_pairs - 1) * 2, False, True)
        wait_stores(t_a, ssem_a)
        wait_stores(t_b, ssem_b)

    return gather_kernel


@jax.jit
def kernel(x, table):
    n_seq, n_pos = x.shape
    n_i, n_j = n_seq, n_pos
    flat = x.T.reshape(-1).astype(jnp.int32)
    out = _make_gather(flat.shape[0], n_i)(flat, table)
    out = out.reshape(n_j, 8, n_i // 128, 8, 128)
    return out.transpose(2, 4, 0, 1, 3).reshape(n_i, n_j, D_MODEL)


# speedup vs baseline: 1.0782x; 1.0061x over previous
import functools

import jax
import jax.numpy as jnp
from jax import lax
from jax.experimental import pallas as pl
from jax.experimental.pallas import tpu as pltpu
from jax.experimental.pallas import tpu_sc as plsc

VOCAB = 1000000
D_MODEL = 64

_info = plsc.get_sparse_core_info()
_NC, _NS = _info.num_cores, _info.num_subcores
_NW = _NC * _NS

_C = 256
_TROWS = 24
_TPITCH = 136
_TBYTES = 8 * 16 * 128 * 4


def _make_gather(B: int, n_i: int):
    assert B % (_NW * _C * 2) == 0 and n_i % _C == 0
    b_per_w = B // _NW
    n_chunks = b_per_w // _C
    n_pairs = n_chunks // 2
    it_per_i = n_i // 128
    mesh = plsc.VectorSubcoreMesh(core_axis_name="c", subcore_axis_name="s")

    @functools.partial(
        pl.kernel,
        out_type=jax.ShapeDtypeStruct((B // 128 * 64, 128), jnp.float32),
        mesh=mesh,
        scratch_types=[
            pltpu.VMEM((b_per_w,), jnp.int32),
            pltpu.VMEM((_C, D_MODEL), jnp.float32),
            pltpu.VMEM((_C, D_MODEL), jnp.float32),
            pltpu.VMEM((8, _TROWS, _TPITCH), jnp.float32),
            pltpu.VMEM((8, _TROWS, _TPITCH), jnp.float32),
            pltpu.SemaphoreType.DMA,
            pltpu.SemaphoreType.DMA,
            pltpu.SemaphoreType.DMA,
            pltpu.SemaphoreType.DMA,
        ],
        compiler_params=pltpu.CompilerParams(
            use_tc_tiling_on_sc=False, needs_layout_passes=False
        ),
    )
    def gather_kernel(idx_hbm, table_hbm, out_hbm, idx_v, rb_a, rb_b, t_a, t_b,
                      gsem_a, gsem_b, ssem_a, ssem_b):
        wid = lax.axis_index("s") * _NC + lax.axis_index("c")
        base = wid * b_per_w

        def start_gather(c, rb, gsem):
            pltpu.async_copy(
                table_hbm.at[idx_v.at[pl.ds(c * _C, _C)]], rb, gsem
            )

        def wait_gather(rb, gsem):
            pltpu.make_async_copy(
                table_hbm.at[idx_v.at[pl.ds(0, _C)]], rb, gsem
            ).wait()

        lanes = lax.iota(jnp.int32, 16)
        kg_vs = [lanes // 8 + 2 * q for q in range(4)]
        ks_v = lanes % 8

        def transpose(rb, t):
            for it in range(_C // 128):
                r_v = ks_v + it * 8

                def rows(cl, carry):
                    for dc in range(4):
                        c = cl * 4 + dc
                        il = jnp.full((16,), c, jnp.int32)
                        for q in range(4):
                            v = rb[it * 128 + c, pl.ds(q * 16, 16)]
                            plsc.store_scatter(t, [kg_vs[q], r_v, il], v)
                    return carry

                lax.fori_loop(0, 32, rows, 0)

        def start_store(c, t, ssem):
            n0 = base + c * _C
            j = n0 // n_i
            it0 = (n0 - j * n_i) // 128
            for kg in range(8):
                r0 = (j * 8 + kg) * it_per_i * 8 + it0 * 8
                pltpu.async_copy(
                    t.at[kg, pl.ds(0, 16), pl.ds(0, 128)],
                    out_hbm.at[pl.ds(r0, 16)],
                    ssem,
                )

        def wait_store(t, ssem):
            pltpu.make_async_copy(
                t.at[0, pl.ds(0, 16), pl.ds(0, 128)],
                out_hbm.at[pl.ds(0, 16)],
                ssem,
            ).wait()

        def wait_stores(t, ssem):
            for kg in range(8):
                wait_store(t, ssem)

        def do_pair(c, issue_next, wait_prev):
            wait_gather(rb_a, gsem_a)
            if wait_prev:
                wait_stores(t_a, ssem_a)
            transpose(rb_a, t_a)
            if issue_next:
                start_gather(c + 2, rb_a, gsem_a)
            start_store(c, t_a, ssem_a)
            wait_gather(rb_b, gsem_b)
            if wait_prev:
                wait_stores(t_b, ssem_b)
            transpose(rb_b, t_b)
            if issue_next:
                start_gather(c + 3, rb_b, gsem_b)
            start_store(c + 1, t_b, ssem_b)

        pltpu.sync_copy(idx_hbm.at[pl.ds(base, b_per_w)], idx_v)

        start_gather(0, rb_a, gsem_a)
        start_gather(1, rb_b, gsem_b)
        do_pair(0, True, False)

        def outer(p, carry):
            do_pair(p * 2, True, True)
            return carry

        lax.fori_loop(1, n_pairs - 1, outer, 0)
        do_pair((n_pairs - 1) * 2, False, True)
        wait_stores(t_a, ssem_a)
        wait_stores(t_b, ssem_b)

    return gather_kernel


@jax.jit
def kernel(x, table):
    n_seq, n_pos = x.shape
    n_i, n_j = n_seq, n_pos
    flat = x.T.reshape(-1).astype(jnp.int32)
    out = _make_gather(flat.shape[0], n_i)(flat, table)
    out = out.reshape(n_j, 8, n_i // 128, 8, 128)
    return out.transpose(2, 4, 0, 1, 3).reshape(n_i, n_j, D_MODEL)
